# baseline (speedup 1.0000x reference)
import functools

import jax
import jax.numpy as jnp
from jax import lax
from jax.experimental import pallas as pl
from jax.experimental.pallas import tpu as pltpu
from jax.experimental.pallas import tpu_sc as plsc

EMBED_DIM = 128
HIST = 50
BATCH = 4096
VOCAB = 1002
NUM_WORKERS = 32
BPW = BATCH // NUM_WORKERS
STEPS = HIST
NBUF = 5
LOOK = 3

_mesh = plsc.VectorSubcoreMesh(core_axis_name="c", subcore_axis_name="s")


@functools.partial(
    pl.kernel,
    mesh=_mesh,
    out_type=jax.ShapeDtypeStruct((HIST, BATCH, EMBED_DIM), jnp.float32),
    scratch_types=[
        pltpu.VMEM((HIST, BPW), jnp.int32),
        pltpu.VMEM((NBUF, BPW, EMBED_DIM), jnp.float32),
        pltpu.VMEM_SHARED((VOCAB, EMBED_DIM), jnp.float32),
        pltpu.SemaphoreType.DMA((NBUF,)),
        pltpu.SemaphoreType.DMA((NBUF,)),
    ],
)
def _sc_gather(idx_hbm, table_hbm, out_hbm, idx_v, bufs, table_sh, gsem, wsem):
    c = lax.axis_index("c")
    s = lax.axis_index("s")
    wid = s * 2 + c
    b0 = wid * BPW

    @pl.when(s == 0)
    def _():
        pltpu.sync_copy(table_hbm, table_sh)

    pltpu.sync_copy(idx_hbm.at[:, pl.ds(b0, BPW)], idx_v)
    plsc.subcore_barrier()

    def gstart(j, b):
        pltpu.async_copy(table_sh.at[idx_v.at[j]], bufs.at[b], gsem.at[b])

    def gwait(j, b):
        pltpu.make_async_copy(
            table_sh.at[idx_v.at[j]], bufs.at[b], gsem.at[b]
        ).wait()

    def wstart(j, b):
        pltpu.async_copy(
            bufs.at[b], out_hbm.at[j, pl.ds(b0, BPW)], wsem.at[b]
        )

    def wwait(j, b):
        pltpu.make_async_copy(
            bufs.at[b], out_hbm.at[j, pl.ds(b0, BPW)], wsem.at[b]
        ).wait()

    for t in range(LOOK):
        gstart(t, t)

    def step(j, t):
        bn = (t + LOOK) % NBUF
        nj = j + LOOK

        @pl.when(jnp.logical_and(nj < STEPS, j >= NBUF - LOOK))
        def _():
            wwait(nj - NBUF, bn)

        @pl.when(nj < STEPS)
        def _():
            gstart(nj, bn)

        gwait(j, t)
        wstart(j, t)

    def body(jj, carry):
        j0 = jj * NBUF
        for t in range(NBUF):
            step(j0 + t, t)
        return carry

    lax.fori_loop(0, STEPS // NBUF, body, 0)
    for j in range((STEPS // NBUF) * NBUF, STEPS):
        step(j, j % NBUF)

    for t in range(NBUF):
        j = STEPS - NBUF + t
        wwait(j, j % NBUF)


def kernel(indices, embed_weight):
    idx_t = indices.astype(jnp.int32).T
    out = _sc_gather(idx_t, embed_weight)
    return out.transpose(1, 0, 2)
